# baseline (speedup 1.0000x reference)
import dataclasses

import jax
import jax.numpy as jnp
from jax import lax
from jax.experimental import pallas as pl
from jax.experimental.pallas import tpu as pltpu
from jax.experimental.pallas import tpu_sc as plsc

D_MODEL = 1024
NUM_TILES = 32
GROUP = 16


def _sc_lookup(table_rep, bits2d, n):
    per_tile = n // NUM_TILES
    ngroups = per_tile // GROUP
    mesh = plsc.VectorSubcoreMesh(core_axis_name="c", subcore_axis_name="s")
    cp = pltpu.CompilerParams()
    if "needs_layout_passes" in pltpu.CompilerParams.__dataclass_fields__:
        cp = dataclasses.replace(cp, needs_layout_passes=False)

    @pl.kernel(
        out_type=jax.ShapeDtypeStruct((n, 8, D_MODEL // 8), table_rep.dtype),
        mesh=mesh,
        scratch_types=[
            pltpu.VMEM((1, per_tile), jnp.int32),
            pltpu.VMEM((2 * GROUP, 8, D_MODEL // 8), jnp.float32),
            pltpu.SemaphoreType.DMA,
            pltpu.SemaphoreType.DMA,
        ],
        compiler_params=cp,
    )
    def lookup_kernel(table_hbm, bits_hbm, out_hbm, bits_v, rows_v, gsem, ssem):
        c = lax.axis_index("c")
        s = lax.axis_index("s")
        t = c * 16 + s
        pltpu.async_copy(bits_hbm.at[pl.ds(t, 1), :], bits_v, gsem).wait()

        idx0 = jnp.zeros((GROUP,), jnp.int32) + 2 * t
        cp0 = pltpu.make_async_copy(
            table_hbm.at[idx0], rows_v.at[pl.ds(0, GROUP)], gsem
        )
        cp1 = pltpu.make_async_copy(
            table_hbm.at[idx0 + 1], rows_v.at[pl.ds(GROUP, GROUP)], gsem
        )
        cp0.start()
        cp1.start()
        cp0.wait()
        cp1.wait()

        row_base = t * per_tile

        @pl.loop(0, ngroups)
        def _(g):
            bits = bits_v[0, pl.ds(g * GROUP, GROUP)]
            pos = lax.iota(jnp.int32, GROUP) + (row_base + g * GROUP)
            _, pos_sorted = plsc.sort_key_val(bits, pos)
            cnt1 = jnp.sum(bits)
            src = rows_v.at[pl.ds(cnt1, GROUP)]
            pltpu.async_copy(src, out_hbm.at[pos_sorted], ssem)

        @pl.loop(0, ngroups)
        def _(g):
            dummy = lax.iota(jnp.int32, GROUP) + row_base
            pltpu.make_async_copy(
                rows_v.at[pl.ds(0, GROUP)], out_hbm.at[dummy], ssem
            ).wait()

    return lookup_kernel(table_rep, bits2d)


def kernel(mask_bits, table):
    b, s = mask_bits.shape
    n = b * s
    bits = mask_bits.astype(jnp.int32).reshape(NUM_TILES, n // NUM_TILES)
    table_rep = jnp.tile(table, (NUM_TILES, 1)).reshape(-1, 8, D_MODEL // 8)
    out = _sc_lookup(table_rep, bits, n)
    return out.reshape(b, s, D_MODEL)
